# Initial kernel scaffold; baseline (speedup 1.0000x reference)
import jax
import jax.numpy as jnp
from jax.experimental import pallas as pl


def kernel(drug, drug_mat, drug_mask, protein, prot_mat, prot_mask, params):
    raise NotImplementedError("write your pallas kernel here")


# speedup vs baseline: 1.3221x; 1.3221x over previous
import numpy as np
import jax
import jax.numpy as jnp
from jax.experimental import pallas as pl
from jax.experimental.pallas import tpu as pltpu

_SCALE = np.float32(np.sqrt(0.5))
_BN_INV = np.float32(1.0 / np.sqrt(1.0 + 1e-5))
_F32 = jnp.float32

_INTERPRET = False


def _dot(a, b):
    return jax.lax.dot_general(a, b, (((1,), (0,)), ((), ())),
                               preferred_element_type=_F32)


def _ln_rows(x, g, b, eps=np.float32(1e-5)):
    m = jnp.mean(x, axis=-1, keepdims=True)
    v = jnp.mean((x - m) * (x - m), axis=-1, keepdims=True)
    return (x - m) * jax.lax.rsqrt(v + eps) * g + b


def _elu(x):
    return jnp.where(x > 0, x, jnp.exp(jnp.minimum(x, 0)) - 1.0)


def _encode(x, fcw, fcb, cw_ref, cb_ref, lng, lnb, L):
    h = _dot(x, fcw) + fcb
    z3 = jnp.zeros((3, 128), _F32)
    for i in range(3):
        hp = jnp.concatenate([z3, h, z3], axis=0)
        c = _dot(hp[0:L, :], cw_ref[i, 0])
        for k in range(1, 7):
            c = c + _dot(hp[k:k + L, :], cw_ref[i, k])
        c = c + cb_ref[i]
        a = c[:, 0:128]
        g = c[:, 128:256]
        h = (a * jax.nn.sigmoid(g) + h) * _SCALE
    pool = jnp.max(h, axis=0, keepdims=True)
    hm = _ln_rows(h, lng, lnb)
    return hm, pool


def _mega_kernel(dfeat, pfeat,
                 dfcw, dfcb, dcw, dcb, dlng, dlnb,
                 pfcw, pfcb, pcw, pcb, plng, plnb,
                 vw, vb, qw, qb, hmat, hbias, poolm,
                 out_logits, out_dpool, out_ppool):
    d_emb, d_pool = _encode(dfeat[0], dfcw[...], dfcb[...], dcw, dcb,
                            dlng[...], dlnb[...], 100)
    p_emb, p_pool = _encode(pfeat[0], pfcw[...], pfcb[...], pcw, pcb,
                            plng[...], plnb[...], 512)
    v_ = jax.nn.relu(_dot(d_emb, vw[...]) + vb[...])
    q_ = jax.nn.relu(_dot(p_emb, qw[...]) + qb[...])
    q_t = q_.T
    logits = jnp.zeros((1, 256), _F32)
    for i in range(2):
        vh = v_ * hmat[i][None, :]
        att = _dot(vh, q_t) + hbias[0, i]
        att = att - jnp.max(att)
        e = jnp.exp(att)
        pm = e * (1.0 / jnp.sum(e))
        t = v_ * _dot(pm, q_)
        f = jnp.sum(t, axis=0, keepdims=True)
        logits = logits + _dot(f, poolm[...])
    out_logits[0] = logits
    out_dpool[0] = d_pool
    out_ppool[0] = p_pool


def _head_kernel(lg, dp, pp, bng, bnb, prew, preb, bn1g, bn1b, postw, postb,
                 g1w, g1b, g1g, g1bb, g2w, g2b, g2g, g2bb, g3w, g3b,
                 ew1, eb1, elng, elnb, ew2, eb2,
                 out_routing, out_oe):
    h = lg[...] * _BN_INV * bng[...] + bnb[...]
    cat = jnp.concatenate([dp[...], pp[...]], axis=1)
    h_pre = _elu(_dot(cat, prew[...]) + preb[...])
    h_pre = h_pre * _BN_INV * bn1g[...] + bn1b[...]
    h_post = _elu(_dot(h, postw[...]) + postb[...])
    x = h_pre + h_post
    g = jax.nn.relu(_ln_rows(_dot(x, g1w[...]) + g1b[...], g1g[...], g1bb[...]))
    g = jax.nn.relu(_ln_rows(_dot(g, g2w[...]) + g2b[...], g2g[...], g2bb[...]))
    gl = _dot(g, g3w[...]) + g3b[...]
    gl = gl - jnp.max(gl, axis=1, keepdims=True)
    eg = jnp.exp(gl)
    out_routing[...] = eg / jnp.sum(eg, axis=1, keepdims=True)
    oes = []
    for e_i in range(4):
        h1 = _dot(x, ew1[e_i]) + eb1[e_i]
        h1 = _elu(_ln_rows(h1, elng[e_i], elnb[e_i]))
        oe = jnp.sum(h1 * ew2[e_i], axis=1, keepdims=True) + eb2[0, e_i]
        oes.append(oe)
    out_oe[...] = jnp.concatenate(oes, axis=1)


def _combine_kernel(r_ref, oe_ref, out_f):
    r = r_ref[...]
    oe = oe_ref[...]
    B = r.shape[0]
    ii = jax.lax.broadcasted_iota(jnp.int32, (B, 4), 1)
    m1 = jnp.max(r, axis=1, keepdims=True)
    i1 = jnp.min(jnp.where(r == m1, ii, 4), axis=1, keepdims=True)
    rm = jnp.where(ii == i1, jnp.float32(-1e30), r)
    m2 = jnp.max(rm, axis=1, keepdims=True)
    i2 = jnp.min(jnp.where(rm == m2, ii, 4), axis=1, keepdims=True)
    g1 = jnp.sum(jnp.where(ii == i1, oe, 0.0), axis=1, keepdims=True)
    g2 = jnp.sum(jnp.where(ii == i2, oe, 0.0), axis=1, keepdims=True)
    e2 = jnp.exp(m2 - m1)
    out_f[...] = (g1 + g2 * e2) / (1.0 + e2)


def _full_spec(shape):
    nd = len(shape)
    return pl.BlockSpec(shape, lambda *_, _n=nd: (0,) * _n)


def kernel(drug, drug_mat, drug_mask, protein, prot_mat, prot_mask, params):
    p = params
    B = drug_mat.shape[0]
    dpr, ppr, ban = p['drug'], p['prot'], p['ban']

    def prep_enc(q):
        return (q['fc_w'], q['fc_b'].reshape(1, -1),
                jnp.transpose(q['conv_w'], (0, 3, 2, 1)),
                q['conv_b'].reshape(3, 1, -1),
                q['ln_g'].reshape(1, -1), q['ln_b'].reshape(1, -1))

    denc = prep_enc(dpr)
    penc = prep_enc(ppr)
    pool_mat = (jnp.arange(768, dtype=jnp.int32)[:, None] // 3
                == jnp.arange(256, dtype=jnp.int32)[None, :]).astype(_F32)
    ban_args = (ban['v_w'], ban['v_b'].reshape(1, -1),
                ban['q_w'], ban['q_b'].reshape(1, -1),
                ban['h_mat'], ban['h_bias'].reshape(1, -1), pool_mat)

    mega_in = (drug_mat, prot_mat) + denc + penc + ban_args
    in_specs = [
        pl.BlockSpec((1,) + drug_mat.shape[1:], lambda b: (b, 0, 0)),
        pl.BlockSpec((1,) + prot_mat.shape[1:], lambda b: (b, 0, 0)),
    ] + [_full_spec(a.shape) for a in mega_in[2:]]

    logits, d_pool, p_pool = pl.pallas_call(
        _mega_kernel,
        grid=(B,),
        in_specs=in_specs,
        out_specs=[
            pl.BlockSpec((1, 1, 256), lambda b: (b, 0, 0)),
            pl.BlockSpec((1, 1, 128), lambda b: (b, 0, 0)),
            pl.BlockSpec((1, 1, 128), lambda b: (b, 0, 0)),
        ],
        out_shape=[
            jax.ShapeDtypeStruct((B, 1, 256), _F32),
            jax.ShapeDtypeStruct((B, 1, 128), _F32),
            jax.ShapeDtypeStruct((B, 1, 128), _F32),
        ],
        compiler_params=pltpu.CompilerParams(
            dimension_semantics=("arbitrary",)),
        interpret=_INTERPRET,
    )(*mega_in)
    logits = logits.reshape(B, 256)
    d_pool = d_pool.reshape(B, 128)
    p_pool = p_pool.reshape(B, 128)

    head_in = (logits, d_pool, p_pool,
               ban['bn_g'].reshape(1, -1), ban['bn_b'].reshape(1, -1),
               p['pre_w'], p['pre_b'].reshape(1, -1),
               p['bn1024_g'].reshape(1, -1), p['bn1024_b'].reshape(1, -1),
               p['post_w'], p['post_b'].reshape(1, -1),
               p['g1_w'], p['g1_b'].reshape(1, -1),
               p['g1_ln_g'].reshape(1, -1), p['g1_ln_b'].reshape(1, -1),
               p['g2_w'], p['g2_b'].reshape(1, -1),
               p['g2_ln_g'].reshape(1, -1), p['g2_ln_b'].reshape(1, -1),
               p['g3_w'], p['g3_b'].reshape(1, -1),
               p['e_w1'], p['e_b1'].reshape(4, 1, -1),
               p['e_ln_g'].reshape(4, 1, -1), p['e_ln_b'].reshape(4, 1, -1),
               jnp.transpose(p['e_w2'], (0, 2, 1)),
               p['e_b2'].reshape(1, -1))

    routing, out_e = pl.pallas_call(
        _head_kernel,
        in_specs=[_full_spec(a.shape) for a in head_in],
        out_specs=[pl.BlockSpec((B, 4), lambda: (0, 0)),
                   pl.BlockSpec((B, 4), lambda: (0, 0))],
        out_shape=[jax.ShapeDtypeStruct((B, 4), _F32),
                   jax.ShapeDtypeStruct((B, 4), _F32)],
        interpret=_INTERPRET,
    )(*head_in)

    final = pl.pallas_call(
        _combine_kernel,
        in_specs=[_full_spec((B, 4)), _full_spec((B, 4))],
        out_specs=pl.BlockSpec((B, 1), lambda: (0, 0)),
        out_shape=jax.ShapeDtypeStruct((B, 1), _F32),
        interpret=_INTERPRET,
    )(routing, out_e)

    return final, routing
